# baseline (speedup 1.0000x reference)
import functools

import jax
import jax.numpy as jnp
from jax import lax
from jax.experimental import pallas as pl
from jax.experimental.pallas import tpu as pltpu
from jax.experimental.pallas import tpu_sc as plsc

N = 10000
NP = 10112
E = 320000
D = 128
NC = 2
NS = 16
LANES = 16
NW = NC * NS

_MESH = plsc.VectorSubcoreMesh(
    core_axis_name="c", subcore_axis_name="s", num_cores=NC, num_subcores=NS
)
_SC_PARAMS = pltpu.CompilerParams(needs_layout_passes=False)

_f32 = jnp.float32
_i32 = jnp.int32


@functools.partial(
    pl.kernel,
    out_type=jax.ShapeDtypeStruct((NW * NP,), _f32),
    mesh=_MESH,
    compiler_params=_SC_PARAMS,
    scratch_types=[
        pltpu.VMEM((NP,), _f32),
        pltpu.VMEM((E // NW,), _i32),
        pltpu.SemaphoreType.DMA,
    ],
)
def _deg_kernel(ei_hbm, degp_hbm, deg_loc, dst_loc, sem_d):
    c = lax.axis_index("c")
    s = lax.axis_index("s")
    w = c * NS + s
    ept = E // NW
    dd = pltpu.async_copy(ei_hbm.at[pl.ds(E + w * ept, ept)], dst_loc, sem_d)
    ones = jnp.ones((LANES,), _f32)
    zvec = jnp.zeros((LANES,), _f32)

    @plsc.parallel_loop(0, NP // LANES, unroll=8)
    def _(i):
        deg_loc[pl.ds(i * LANES, LANES)] = zvec

    dd.wait()

    @plsc.parallel_loop(0, ept // LANES, unroll=8)
    def _(i):
        dv = dst_loc[pl.ds(i * LANES, LANES)]
        plsc.addupdate_scatter(deg_loc, [dv], ones)

    pltpu.sync_copy(deg_loc, degp_hbm.at[pl.ds(w * NP, NP)])


def _make_agg(fo, g, chunk):
    ngroups = fo // g
    ns = NW // ngroups
    ept = E // ns
    nchunks = ept // chunk
    row = g * NP
    nb = 4
    assert ept % chunk == 0 and chunk % LANES == 0 and nchunks >= nb

    @functools.partial(
        pl.kernel,
        out_type=jax.ShapeDtypeStruct((ns * fo * NP,), _f32),
        mesh=_MESH,
        compiler_params=_SC_PARAMS,
        scratch_types=[
            pltpu.VMEM((row,), _f32),
            pltpu.VMEM((row,), _f32),
            pltpu.VMEM((chunk,), _i32),
            pltpu.VMEM((chunk,), _i32),
            pltpu.VMEM((chunk,), _i32),
            pltpu.VMEM((chunk,), _i32),
            pltpu.VMEM((chunk,), _i32),
            pltpu.VMEM((chunk,), _i32),
            pltpu.VMEM((chunk,), _i32),
            pltpu.VMEM((chunk,), _i32),
            pltpu.SemaphoreType.DMA,
            pltpu.SemaphoreType.DMA,
            pltpu.SemaphoreType.DMA,
            pltpu.SemaphoreType.DMA,
            pltpu.SemaphoreType.DMA,
        ],
    )
    def agg(hp_hbm, ei_hbm, part_hbm,
            h_loc, acc_loc, s0, s1, s2, s3, d0, d1, d2, d3,
            sem_h, sem0, sem1, sem2, sem3):
        c = lax.axis_index("c")
        s = lax.axis_index("s")
        w = c * NS + s
        grp = w // ns
        sl = w % ns
        ebase = sl * ept
        sbufs, dbufs = [s0, s1, s2, s3], [d0, d1, d2, d3]
        sems = [sem0, sem1, sem2, sem3]

        hz = pltpu.async_copy(hp_hbm.at[pl.ds(grp * row, row)], h_loc, sem_h)

        def start(k, b):
            base = ebase + k * chunk
            pltpu.async_copy(ei_hbm.at[pl.ds(base, chunk)], sbufs[b], sems[b])
            pltpu.async_copy(ei_hbm.at[pl.ds(E + base, chunk)], dbufs[b], sems[b])

        def wait(b):
            pltpu.make_async_copy(ei_hbm.at[pl.ds(0, chunk)], sbufs[b], sems[b]).wait()
            pltpu.make_async_copy(ei_hbm.at[pl.ds(0, chunk)], dbufs[b], sems[b]).wait()

        for b in range(nb):
            start(b, b)

        zvec = jnp.zeros((LANES,), _f32)

        @plsc.parallel_loop(0, row // LANES, unroll=8)
        def _(i):
            acc_loc[pl.ds(i * LANES, LANES)] = zvec

        hz.wait()

        def process(b):
            @plsc.parallel_loop(0, chunk // LANES, unroll=16)
            def _(i):
                sv = sbufs[b][pl.ds(i * LANES, LANES)]
                dv = dbufs[b][pl.ds(i * LANES, LANES)]
                for f in range(g):
                    so = sv + (f * NP) if f else sv
                    do = dv + (f * NP) if f else dv
                    vals = plsc.load_gather(h_loc, [so])
                    plsc.addupdate_scatter(acc_loc, [do], vals)

        def step(j, carry):
            for b in range(nb):
                k = j * nb + b
                wait(b)
                process(b)

                @pl.when(k + nb < nchunks)
                def _():
                    start(k + nb, b)

            return carry

        lax.fori_loop(0, nchunks // nb, step, 0)
        for r in range(nchunks % nb):
            b = r % nb
            wait(b)
            process(b)
        pltpu.sync_copy(acc_loc, part_hbm.at[pl.ds((sl * fo + grp * g) * NP, row)])

    return agg


_agg8 = _make_agg(8, 4, 2000)
_agg16 = _make_agg(16, 4, 2000)
_agg2 = _make_agg(2, 1, 2000)


def _row(ref, r):
    return ref[pl.ds(r * NP, N)]


def _reduce_rows(part_ref, fo, ns):
    rows = []
    for f in range(fo):
        acc = _row(part_ref, f)
        for sli in range(1, ns):
            acc = acc + _row(part_ref, sli * fo + f)
        rows.append(acc[None, :])
    return jnp.concatenate(rows, axis=0)


def _stack_rows(ref, fo):
    return jnp.concatenate([_row(ref, f)[None, :] for f in range(fo)], axis=0)


def _store_rows(ref, vals, fo):
    for f in range(fo):
        ref[pl.ds(f * NP, N)] = vals[f]


def _mm1_body(x_ref, w1_ref, h_ref):
    h_ref[...] = lax.dot_general(w1_ref[...], x_ref[...], (((0,), (1,)), ((), ())),
                                 preferred_element_type=_f32)


def _prep_body(degp_ref, h_ref, dinv_ref, hp_ref):
    deg = _row(degp_ref, 0)
    for wn in range(1, NW):
        deg = deg + _row(degp_ref, wn)
    dinv = lax.rsqrt(deg + 1.0)[None, :]
    dinv_ref[...] = dinv
    _store_rows(hp_ref, h_ref[...] * dinv, 8)


def _make_mid(fo, ns):
    def mid_body(part_ref, hp_ref, dinv_ref, b_ref, w_ref, out_ref):
        dinv = dinv_ref[...]
        agg = _reduce_rows(part_ref, fo, ns)
        z = dinv * (agg + _stack_rows(hp_ref, fo)) + b_ref[...]
        act = jnp.where(z > 0, z, jnp.exp(z) - 1.0)
        h = lax.dot_general(w_ref[...], act, (((0,), (0,)), ((), ())),
                            preferred_element_type=_f32)
        hv = h * dinv
        _store_rows(out_ref, hv, hv.shape[0])
    return mid_body


def _fin_body(part_ref, hp_ref, dinv_ref, b_ref, out_ref):
    agg = _reduce_rows(part_ref, 2, 16)
    z = dinv_ref[...] * (agg + _stack_rows(hp_ref, 2)) + b_ref[...]
    m = jnp.max(z, axis=0, keepdims=True)
    out_ref[...] = z - (jnp.log(jnp.sum(jnp.exp(z - m), axis=0, keepdims=True)) + m)


def _tc(body, out_shapes):
    return pl.pallas_call(body, out_shape=out_shapes)


def _flat(n):
    return jax.ShapeDtypeStruct((n * NP,), _f32)


def kernel(x, edge_index, laplacian_index, laplacian_weight,
           W1, b1, W2, b2, W3, b3, W4, b4):
    del laplacian_index, laplacian_weight
    ei = edge_index.reshape(-1)

    degp = _deg_kernel(ei)
    h1 = _tc(_mm1_body, jax.ShapeDtypeStruct((8, N), _f32))(x, W1)
    dinv, hp1 = _tc(_prep_body, [
        jax.ShapeDtypeStruct((1, N), _f32), _flat(8),
    ])(degp, h1)

    p1 = _agg8(hp1, ei)
    hp2 = _tc(_make_mid(8, 16), _flat(16))(p1, hp1, dinv, b1.reshape(8, 1), W2)

    p2 = _agg16(hp2, ei)
    hp3 = _tc(_make_mid(16, 8), _flat(8))(p2, hp2, dinv, b2.reshape(16, 1), W3)

    p3 = _agg8(hp3, ei)
    hp4 = _tc(_make_mid(8, 16), _flat(2))(p3, hp3, dinv, b3.reshape(8, 1), W4)

    p4 = _agg2(hp4, ei)
    out = _tc(_fin_body, jax.ShapeDtypeStruct((2, N), _f32))(
        p4, hp4, dinv, b4.reshape(2, 1))

    return out.T


# speedup vs baseline: 1.1455x; 1.1455x over previous
import functools

import jax
import jax.numpy as jnp
from jax import lax
from jax.experimental import pallas as pl
from jax.experimental.pallas import tpu as pltpu
from jax.experimental.pallas import tpu_sc as plsc

N = 10000
NP = 10112
E = 320000
D = 128
NC = 2
NS = 16
LANES = 16
NW = NC * NS

_MESH = plsc.VectorSubcoreMesh(
    core_axis_name="c", subcore_axis_name="s", num_cores=NC, num_subcores=NS
)
_SC_PARAMS = pltpu.CompilerParams(needs_layout_passes=False)

_f32 = jnp.float32
_i32 = jnp.int32


@functools.partial(
    pl.kernel,
    out_type=jax.ShapeDtypeStruct((NW * NP,), _f32),
    mesh=_MESH,
    compiler_params=_SC_PARAMS,
    scratch_types=[
        pltpu.VMEM((NP,), _f32),
        pltpu.VMEM((E // NW,), _i32),
        pltpu.SemaphoreType.DMA,
    ],
)
def _deg_kernel(ei_hbm, degp_hbm, deg_loc, dst_loc, sem_d):
    c = lax.axis_index("c")
    s = lax.axis_index("s")
    w = c * NS + s
    ept = E // NW
    dd = pltpu.async_copy(ei_hbm.at[pl.ds(E + w * ept, ept)], dst_loc, sem_d)
    ones = jnp.ones((LANES,), _f32)
    zvec = jnp.zeros((LANES,), _f32)

    @plsc.parallel_loop(0, NP // LANES, unroll=8)
    def _(i):
        deg_loc[pl.ds(i * LANES, LANES)] = zvec

    dd.wait()

    @plsc.parallel_loop(0, ept // LANES, unroll=8)
    def _(i):
        dv = dst_loc[pl.ds(i * LANES, LANES)]
        plsc.addupdate_scatter(deg_loc, [dv], ones)

    pltpu.sync_copy(deg_loc, degp_hbm.at[pl.ds(w * NP, NP)])


def _make_agg(fo, g, chunk):
    ngroups = fo // g
    ns = NW // ngroups
    ept = E // ns
    nchunks = ept // chunk
    row = g * NP
    nb = 4
    assert ept % chunk == 0 and chunk % LANES == 0 and nchunks >= nb

    @functools.partial(
        pl.kernel,
        out_type=jax.ShapeDtypeStruct((ns * fo * NP,), _f32),
        mesh=_MESH,
        compiler_params=_SC_PARAMS,
        scratch_types=[
            pltpu.VMEM((row,), _f32),
            pltpu.VMEM((row,), _f32),
            pltpu.VMEM((chunk,), _i32),
            pltpu.VMEM((chunk,), _i32),
            pltpu.VMEM((chunk,), _i32),
            pltpu.VMEM((chunk,), _i32),
            pltpu.VMEM((chunk,), _i32),
            pltpu.VMEM((chunk,), _i32),
            pltpu.VMEM((chunk,), _i32),
            pltpu.VMEM((chunk,), _i32),
            pltpu.SemaphoreType.DMA,
            pltpu.SemaphoreType.DMA,
            pltpu.SemaphoreType.DMA,
            pltpu.SemaphoreType.DMA,
            pltpu.SemaphoreType.DMA,
        ],
    )
    def agg(hp_hbm, ei_hbm, part_hbm,
            h_loc, acc_loc, s0, s1, s2, s3, d0, d1, d2, d3,
            sem_h, sem0, sem1, sem2, sem3):
        c = lax.axis_index("c")
        s = lax.axis_index("s")
        w = c * NS + s
        grp = w // ns
        sl = w % ns
        ebase = sl * ept
        sbufs, dbufs = [s0, s1, s2, s3], [d0, d1, d2, d3]
        sems = [sem0, sem1, sem2, sem3]

        hz = pltpu.async_copy(hp_hbm.at[pl.ds(grp * row, row)], h_loc, sem_h)

        def start(k, b):
            base = ebase + k * chunk
            pltpu.async_copy(ei_hbm.at[pl.ds(base, chunk)], sbufs[b], sems[b])
            pltpu.async_copy(ei_hbm.at[pl.ds(E + base, chunk)], dbufs[b], sems[b])

        def wait(b):
            pltpu.make_async_copy(ei_hbm.at[pl.ds(0, chunk)], sbufs[b], sems[b]).wait()
            pltpu.make_async_copy(ei_hbm.at[pl.ds(0, chunk)], dbufs[b], sems[b]).wait()

        for b in range(nb):
            start(b, b)

        zvec = jnp.zeros((LANES,), _f32)

        @plsc.parallel_loop(0, row // LANES, unroll=8)
        def _(i):
            acc_loc[pl.ds(i * LANES, LANES)] = zvec

        hz.wait()

        def process(b):
            @plsc.parallel_loop(0, chunk // LANES, unroll=8)
            def _(i):
                sv = sbufs[b][pl.ds(i * LANES, LANES)]
                dv = dbufs[b][pl.ds(i * LANES, LANES)]
                for f in range(g):
                    so = sv + (f * NP) if f else sv
                    do = dv + (f * NP) if f else dv
                    vals = plsc.load_gather(h_loc, [so])
                    plsc.addupdate_scatter(acc_loc, [do], vals)

        def step(j, carry):
            for b in range(nb):
                k = j * nb + b
                wait(b)
                process(b)

                @pl.when(k + nb < nchunks)
                def _():
                    start(k + nb, b)

            return carry

        lax.fori_loop(0, nchunks // nb, step, 0)
        for r in range(nchunks % nb):
            b = r % nb
            wait(b)
            process(b)
        pltpu.sync_copy(acc_loc, part_hbm.at[pl.ds((sl * fo + grp * g) * NP, row)])

    return agg


_agg8 = _make_agg(8, 4, 2000)
_agg16 = _make_agg(16, 4, 2000)
_agg2 = _make_agg(2, 1, 2000)


def _row(ref, r):
    return ref[pl.ds(r * NP, N)]


def _reduce_rows(part_ref, fo, ns):
    rows = []
    for f in range(fo):
        acc = _row(part_ref, f)
        for sli in range(1, ns):
            acc = acc + _row(part_ref, sli * fo + f)
        rows.append(acc[None, :])
    return jnp.concatenate(rows, axis=0)


def _stack_rows(ref, fo):
    return jnp.concatenate([_row(ref, f)[None, :] for f in range(fo)], axis=0)


def _store_rows(ref, vals, fo):
    for f in range(fo):
        ref[pl.ds(f * NP, N)] = vals[f]


def _mm1_body(x_ref, w1_ref, h_ref):
    h_ref[...] = lax.dot_general(w1_ref[...], x_ref[...], (((0,), (1,)), ((), ())),
                                 preferred_element_type=_f32)


def _prep_body(degp_ref, h_ref, dinv_ref, hp_ref):
    deg = _row(degp_ref, 0)
    for wn in range(1, NW):
        deg = deg + _row(degp_ref, wn)
    dinv = lax.rsqrt(deg + 1.0)[None, :]
    dinv_ref[...] = dinv
    _store_rows(hp_ref, h_ref[...] * dinv, 8)


def _make_mid(fo, ns):
    def mid_body(part_ref, hp_ref, dinv_ref, b_ref, w_ref, out_ref):
        dinv = dinv_ref[...]
        agg = _reduce_rows(part_ref, fo, ns)
        z = dinv * (agg + _stack_rows(hp_ref, fo)) + b_ref[...]
        act = jnp.where(z > 0, z, jnp.exp(z) - 1.0)
        h = lax.dot_general(w_ref[...], act, (((0,), (0,)), ((), ())),
                            preferred_element_type=_f32)
        hv = h * dinv
        _store_rows(out_ref, hv, hv.shape[0])
    return mid_body


def _fin_body(part_ref, hp_ref, dinv_ref, b_ref, out_ref):
    agg = _reduce_rows(part_ref, 2, 16)
    z = dinv_ref[...] * (agg + _stack_rows(hp_ref, 2)) + b_ref[...]
    m = jnp.max(z, axis=0, keepdims=True)
    out_ref[...] = z - (jnp.log(jnp.sum(jnp.exp(z - m), axis=0, keepdims=True)) + m)


def _tc(body, out_shapes):
    return pl.pallas_call(body, out_shape=out_shapes)


def _flat(n):
    return jax.ShapeDtypeStruct((n * NP,), _f32)


def kernel(x, edge_index, laplacian_index, laplacian_weight,
           W1, b1, W2, b2, W3, b3, W4, b4):
    del laplacian_index, laplacian_weight
    ei = edge_index.reshape(-1)

    degp = _deg_kernel(ei)
    h1 = _tc(_mm1_body, jax.ShapeDtypeStruct((8, N), _f32))(x, W1)
    dinv, hp1 = _tc(_prep_body, [
        jax.ShapeDtypeStruct((1, N), _f32), _flat(8),
    ])(degp, h1)

    p1 = _agg8(hp1, ei)
    hp2 = _tc(_make_mid(8, 16), _flat(16))(p1, hp1, dinv, b1.reshape(8, 1), W2)

    p2 = _agg16(hp2, ei)
    hp3 = _tc(_make_mid(16, 8), _flat(8))(p2, hp2, dinv, b2.reshape(16, 1), W3)

    p3 = _agg8(hp3, ei)
    hp4 = _tc(_make_mid(8, 16), _flat(2))(p3, hp3, dinv, b3.reshape(8, 1), W4)

    p4 = _agg2(hp4, ei)
    out = _tc(_fin_body, jax.ShapeDtypeStruct((2, N), _f32))(
        p4, hp4, dinv, b4.reshape(2, 1))

    return out.T


# speedup vs baseline: 1.1593x; 1.0121x over previous
import functools

import jax
import jax.numpy as jnp
from jax import lax
from jax.experimental import pallas as pl
from jax.experimental.pallas import tpu as pltpu
from jax.experimental.pallas import tpu_sc as plsc

N = 10000
NP = 10112
E = 320000
D = 128
NC = 2
NS = 16
LANES = 16
NW = NC * NS

_MESH = plsc.VectorSubcoreMesh(
    core_axis_name="c", subcore_axis_name="s", num_cores=NC, num_subcores=NS
)
_SC_PARAMS = pltpu.CompilerParams(needs_layout_passes=False)

_f32 = jnp.float32
_i32 = jnp.int32


@functools.partial(
    pl.kernel,
    out_type=jax.ShapeDtypeStruct((NW * NP,), _f32),
    mesh=_MESH,
    compiler_params=_SC_PARAMS,
    scratch_types=[
        pltpu.VMEM((NP,), _f32),
        pltpu.VMEM((E // NW,), _i32),
        pltpu.SemaphoreType.DMA,
    ],
)
def _deg_kernel(ei_hbm, degp_hbm, deg_loc, dst_loc, sem_d):
    c = lax.axis_index("c")
    s = lax.axis_index("s")
    w = c * NS + s
    ept = E // NW
    dd = pltpu.async_copy(ei_hbm.at[pl.ds(E + w * ept, ept)], dst_loc, sem_d)
    ones = jnp.ones((LANES,), _f32)
    zvec = jnp.zeros((LANES,), _f32)

    @plsc.parallel_loop(0, NP // LANES, unroll=8)
    def _(i):
        deg_loc[pl.ds(i * LANES, LANES)] = zvec

    dd.wait()

    @plsc.parallel_loop(0, ept // LANES, unroll=8)
    def _(i):
        dv = dst_loc[pl.ds(i * LANES, LANES)]
        plsc.addupdate_scatter(deg_loc, [dv], ones)

    pltpu.sync_copy(deg_loc, degp_hbm.at[pl.ds(w * NP, NP)])


def _make_agg(fo, g, chunk):
    ngroups = fo // g
    ns = NW // ngroups
    ept = E // ns
    nchunks = ept // chunk
    row = g * NP
    nb = 4
    nb = min(nb, nchunks)
    assert ept % chunk == 0 and chunk % LANES == 0 and nchunks >= nb

    @functools.partial(
        pl.kernel,
        out_type=jax.ShapeDtypeStruct((ns * fo * NP,), _f32),
        mesh=_MESH,
        compiler_params=_SC_PARAMS,
        scratch_types=[
            pltpu.VMEM((row,), _f32),
            pltpu.VMEM((row,), _f32),
            pltpu.VMEM((chunk,), _i32),
            pltpu.VMEM((chunk,), _i32),
            pltpu.VMEM((chunk,), _i32),
            pltpu.VMEM((chunk,), _i32),
            pltpu.VMEM((chunk,), _i32),
            pltpu.VMEM((chunk,), _i32),
            pltpu.VMEM((chunk,), _i32),
            pltpu.VMEM((chunk,), _i32),
            pltpu.SemaphoreType.DMA,
            pltpu.SemaphoreType.DMA,
            pltpu.SemaphoreType.DMA,
            pltpu.SemaphoreType.DMA,
            pltpu.SemaphoreType.DMA,
        ],
    )
    def agg(hp_hbm, ei_hbm, part_hbm,
            h_loc, acc_loc, s0, s1, s2, s3, d0, d1, d2, d3,
            sem_h, sem0, sem1, sem2, sem3):
        c = lax.axis_index("c")
        s = lax.axis_index("s")
        w = c * NS + s
        grp = w // ns
        sl = w % ns
        ebase = sl * ept
        sbufs, dbufs = [s0, s1, s2, s3], [d0, d1, d2, d3]
        sems = [sem0, sem1, sem2, sem3]

        hz = pltpu.async_copy(hp_hbm.at[pl.ds(grp * row, row)], h_loc, sem_h)

        def start(k, b):
            base = ebase + k * chunk
            pltpu.async_copy(ei_hbm.at[pl.ds(base, chunk)], sbufs[b], sems[b])
            pltpu.async_copy(ei_hbm.at[pl.ds(E + base, chunk)], dbufs[b], sems[b])

        def wait(b):
            pltpu.make_async_copy(ei_hbm.at[pl.ds(0, chunk)], sbufs[b], sems[b]).wait()
            pltpu.make_async_copy(ei_hbm.at[pl.ds(0, chunk)], dbufs[b], sems[b]).wait()

        for b in range(nb):
            start(b, b)

        zvec = jnp.zeros((LANES,), _f32)

        @plsc.parallel_loop(0, row // LANES, unroll=8)
        def _(i):
            acc_loc[pl.ds(i * LANES, LANES)] = zvec

        hz.wait()

        def process(b):
            @plsc.parallel_loop(0, chunk // LANES, unroll=8)
            def _(i):
                sv = sbufs[b][pl.ds(i * LANES, LANES)]
                dv = dbufs[b][pl.ds(i * LANES, LANES)]
                for f in range(g):
                    so = sv + (f * NP) if f else sv
                    do = dv + (f * NP) if f else dv
                    vals = plsc.load_gather(h_loc, [so])
                    plsc.addupdate_scatter(acc_loc, [do], vals)

        def step(j, carry):
            for b in range(nb):
                k = j * nb + b
                wait(b)
                process(b)

                @pl.when(k + nb < nchunks)
                def _():
                    start(k + nb, b)

            return carry

        lax.fori_loop(0, nchunks // nb, step, 0)
        for r in range(nchunks % nb):
            b = r % nb
            wait(b)
            process(b)
        pltpu.sync_copy(acc_loc, part_hbm.at[pl.ds((sl * fo + grp * g) * NP, row)])

    return agg


_agg8 = _make_agg(8, 4, 4000)
_agg16 = _make_agg(16, 4, 4000)
_agg2 = _make_agg(2, 1, 4000)


def _row(ref, r):
    return ref[pl.ds(r * NP, N)]


def _reduce_rows(part_ref, fo, ns):
    rows = []
    for f in range(fo):
        acc = _row(part_ref, f)
        for sli in range(1, ns):
            acc = acc + _row(part_ref, sli * fo + f)
        rows.append(acc[None, :])
    return jnp.concatenate(rows, axis=0)


def _stack_rows(ref, fo):
    return jnp.concatenate([_row(ref, f)[None, :] for f in range(fo)], axis=0)


def _store_rows(ref, vals, fo):
    for f in range(fo):
        ref[pl.ds(f * NP, N)] = vals[f]


def _mm1_body(x_ref, w1_ref, h_ref):
    h_ref[...] = lax.dot_general(w1_ref[...], x_ref[...], (((0,), (1,)), ((), ())),
                                 preferred_element_type=_f32)


def _prep_body(degp_ref, h_ref, dinv_ref, hp_ref):
    deg = _row(degp_ref, 0)
    for wn in range(1, NW):
        deg = deg + _row(degp_ref, wn)
    dinv = lax.rsqrt(deg + 1.0)[None, :]
    dinv_ref[...] = dinv
    _store_rows(hp_ref, h_ref[...] * dinv, 8)


def _make_mid(fo, ns):
    def mid_body(part_ref, hp_ref, dinv_ref, b_ref, w_ref, out_ref):
        dinv = dinv_ref[...]
        agg = _reduce_rows(part_ref, fo, ns)
        z = dinv * (agg + _stack_rows(hp_ref, fo)) + b_ref[...]
        act = jnp.where(z > 0, z, jnp.exp(z) - 1.0)
        h = lax.dot_general(w_ref[...], act, (((0,), (0,)), ((), ())),
                            preferred_element_type=_f32)
        hv = h * dinv
        _store_rows(out_ref, hv, hv.shape[0])
    return mid_body


def _fin_body(part_ref, hp_ref, dinv_ref, b_ref, out_ref):
    agg = _reduce_rows(part_ref, 2, 16)
    z = dinv_ref[...] * (agg + _stack_rows(hp_ref, 2)) + b_ref[...]
    m = jnp.max(z, axis=0, keepdims=True)
    out_ref[...] = z - (jnp.log(jnp.sum(jnp.exp(z - m), axis=0, keepdims=True)) + m)


def _tc(body, out_shapes):
    return pl.pallas_call(body, out_shape=out_shapes)


def _flat(n):
    return jax.ShapeDtypeStruct((n * NP,), _f32)


def kernel(x, edge_index, laplacian_index, laplacian_weight,
           W1, b1, W2, b2, W3, b3, W4, b4):
    del laplacian_index, laplacian_weight
    ei = edge_index.reshape(-1)

    degp = _deg_kernel(ei)
    h1 = _tc(_mm1_body, jax.ShapeDtypeStruct((8, N), _f32))(x, W1)
    dinv, hp1 = _tc(_prep_body, [
        jax.ShapeDtypeStruct((1, N), _f32), _flat(8),
    ])(degp, h1)

    p1 = _agg8(hp1, ei)
    hp2 = _tc(_make_mid(8, 16), _flat(16))(p1, hp1, dinv, b1.reshape(8, 1), W2)

    p2 = _agg16(hp2, ei)
    hp3 = _tc(_make_mid(16, 8), _flat(8))(p2, hp2, dinv, b2.reshape(16, 1), W3)

    p3 = _agg8(hp3, ei)
    hp4 = _tc(_make_mid(8, 16), _flat(2))(p3, hp3, dinv, b3.reshape(8, 1), W4)

    p4 = _agg2(hp4, ei)
    out = _tc(_fin_body, jax.ShapeDtypeStruct((2, N), _f32))(
        p4, hp4, dinv, b4.reshape(2, 1))

    return out.T
